# baseline (speedup 1.0000x reference)
import functools

import jax
import jax.numpy as jnp
from jax import lax
from jax.experimental import pallas as pl
from jax.experimental.pallas import tpu as pltpu
from jax.experimental.pallas import tpu_sc as plsc

_H = 128
_CH = 128
_GCH = 64
_NSC = 2
_NTILE = 16
_NW = _NSC * _NTILE
_BM = 512


def _rup(n, m):
    return ((n + m - 1) // m) * m


def _pad_idx(a, n, val):
    if a.shape[0] == n:
        return a
    return jnp.concatenate([a, jnp.full((n - a.shape[0],), val, a.dtype)])


def _ln_in(o, g, b):
    mu = jnp.mean(o, axis=-1, keepdims=True)
    d = o - mu
    var = jnp.mean(d * d, axis=-1, keepdims=True)
    return d * lax.rsqrt(var + 1e-5) * g + b


def _dot(a, b):
    return jnp.dot(a, b, preferred_element_type=jnp.float32)


def _mlp_ln_body(x_ref, w0_ref, b0_ref, w1_ref, b1_ref, g_ref, bb_ref, o_ref):
    h = jnp.maximum(_dot(x_ref[...], w0_ref[...]) + b0_ref[...], 0.0)
    o = _dot(h, w1_ref[...]) + b1_ref[...]
    o_ref[...] = _ln_in(o, g_ref[...], bb_ref[...])


def _tc_mlp_ln(x, w0, b0, w1, b1, g, b):
    n, k = x.shape
    grid = (n // _BM,)
    return pl.pallas_call(
        _mlp_ln_body,
        grid=grid,
        in_specs=[
            pl.BlockSpec((_BM, k), lambda i: (i, 0)),
            pl.BlockSpec((k, _H), lambda i: (0, 0)),
            pl.BlockSpec((1, _H), lambda i: (0, 0)),
            pl.BlockSpec((_H, _H), lambda i: (0, 0)),
            pl.BlockSpec((1, _H), lambda i: (0, 0)),
            pl.BlockSpec((1, _H), lambda i: (0, 0)),
            pl.BlockSpec((1, _H), lambda i: (0, 0)),
        ],
        out_specs=pl.BlockSpec((_BM, _H), lambda i: (i, 0)),
        out_shape=jax.ShapeDtypeStruct((n, _H), jnp.float32),
    )(x, w0, b0, w1, b1, g, b)


def _edge_enc_body(ep_ref, w0b_ref, b0_ref, w1_ref, b1_ref, g_ref, bb_ref,
                   o_ref):
    h8 = _dot(ep_ref[...], w0b_ref[...])
    h = jnp.reshape(h8, (_BM, _H))
    h = jnp.maximum(h + b0_ref[...], 0.0)
    o = _dot(h, w1_ref[...]) + b1_ref[...]
    o_ref[...] = _ln_in(o, g_ref[...], bb_ref[...])


def _tc_edge_enc(ep, w0big, b0, w1, b1, g, b, n, ofs):
    grid = (n // _BM,)
    nb0 = ofs // _BM
    full = lambda i: (0, 0)
    return pl.pallas_call(
        _edge_enc_body,
        grid=grid,
        in_specs=[
            pl.BlockSpec((_BM // 8, _H), lambda i: (i + nb0, 0)),
            pl.BlockSpec((_H, 8 * _H), full),
            pl.BlockSpec((1, _H), full),
            pl.BlockSpec((_H, _H), full),
            pl.BlockSpec((1, _H), full),
            pl.BlockSpec((1, _H), full),
            pl.BlockSpec((1, _H), full),
        ],
        out_specs=pl.BlockSpec((_BM, _H), lambda i: (i, 0)),
        out_shape=jax.ShapeDtypeStruct((n, _H), jnp.float32),
    )(ep, w0big, b0, w1, b1, g, b)


def _matmul_body(x_ref, w_ref, o_ref):
    o_ref[...] = _dot(x_ref[...], w_ref[...])


def _tc_matmul(x, w):
    n, k = x.shape
    m = w.shape[1]
    grid = (n // _BM,)
    return pl.pallas_call(
        _matmul_body,
        grid=grid,
        in_specs=[
            pl.BlockSpec((_BM, k), lambda i: (i, 0)),
            pl.BlockSpec((k, m), lambda i: (0, 0)),
        ],
        out_specs=pl.BlockSpec((_BM, m), lambda i: (i, 0)),
        out_shape=jax.ShapeDtypeStruct((n, m), jnp.float32),
    )(x, w)


def _edge_step_body(e_ref, gm_ref, gu_ref, w0e_ref, b0_ref, w1_ref, b1_ref,
                    g_ref, bb_ref, msg_ref, en_ref):
    e = e_ref[...]
    ew = _dot(e, w0e_ref[...]) + b0_ref[...]
    hm = jnp.maximum(gm_ref[...] + ew, 0.0)
    hu = jnp.maximum(gu_ref[...] + ew, 0.0)
    g = g_ref[...]
    bb = bb_ref[...]
    w1 = w1_ref[...]
    b1 = b1_ref[...]
    msg_ref[...] = _ln_in(_dot(hm, w1) + b1, g, bb)
    en_ref[...] = _ln_in(_dot(hu, w1) + b1, g, bb) + e


def _tc_edge_step(e, gm, gu, w0e, b0, w1, b1, g, b):
    n = e.shape[0]
    grid = (n // _BM,)
    row = lambda i: (i, 0)
    full = lambda i: (0, 0)
    return pl.pallas_call(
        _edge_step_body,
        grid=grid,
        in_specs=[
            pl.BlockSpec((_BM, _H), row),
            pl.BlockSpec((_BM, _H), row),
            pl.BlockSpec((_BM, _H), row),
            pl.BlockSpec((_H, _H), full),
            pl.BlockSpec((1, _H), full),
            pl.BlockSpec((_H, _H), full),
            pl.BlockSpec((1, _H), full),
            pl.BlockSpec((1, _H), full),
            pl.BlockSpec((1, _H), full),
        ],
        out_specs=[pl.BlockSpec((_BM, _H), row), pl.BlockSpec((_BM, _H), row)],
        out_shape=[jax.ShapeDtypeStruct((n, _H), jnp.float32),
                   jax.ShapeDtypeStruct((n, _H), jnp.float32)],
    )(e, gm, gu, w0e, b0, w1, b1, g, b)


def _make_node_step(nparts):
    def body(*refs):
        parts = refs[:nparts]
        (x_ref, w0a_ref, w0x_ref, b0_ref, w1_ref, b1_ref, g_ref, bb_ref,
         o_ref) = refs[nparts:]
        x = x_ref[...]
        agg = parts[0][...]
        for p in parts[1:]:
            agg = agg + p[...]
        h = jnp.maximum(_dot(agg, w0a_ref[...]) + _dot(x, w0x_ref[...])
                        + b0_ref[...], 0.0)
        o = _dot(h, w1_ref[...]) + b1_ref[...]
        o_ref[...] = _ln_in(o, g_ref[...], bb_ref[...]) + x
    return body


def _tc_node_step(parts, x, w0a, w0x, b0, w1, b1, g, b):
    n = x.shape[0]
    grid = (n // _BM,)
    row = lambda i: (i, 0)
    full = lambda i: (0, 0)
    return pl.pallas_call(
        _make_node_step(len(parts)),
        grid=grid,
        in_specs=[pl.BlockSpec((_BM, _H), row) for _ in parts] + [
            pl.BlockSpec((_BM, _H), row),
            pl.BlockSpec((_H, _H), full),
            pl.BlockSpec((_H, _H), full),
            pl.BlockSpec((1, _H), full),
            pl.BlockSpec((_H, _H), full),
            pl.BlockSpec((1, _H), full),
            pl.BlockSpec((1, _H), full),
            pl.BlockSpec((1, _H), full),
        ],
        out_specs=pl.BlockSpec((_BM, _H), row),
        out_shape=jax.ShapeDtypeStruct((n, _H), jnp.float32),
    )(*parts, x, w0a, w0x, b0, w1, b1, g, b)


def _dec_body(x_ref, cx_ref, w0a_ref, w0x_ref, b0_ref, w1_ref, b1_ref, o_ref):
    h = jnp.maximum(_dot(x_ref[...], w0a_ref[...])
                    + _dot(cx_ref[...], w0x_ref[...]) + b0_ref[...], 0.0)
    o_ref[...] = _dot(h, w1_ref[...]) + b1_ref[...]


def _tc_decoder(x, cx, w0a, w0x, b0, w1, b1):
    n = x.shape[0]
    grid = (n // _BM,)
    row = lambda i: (i, 0)
    full = lambda i: (0, 0)
    return pl.pallas_call(
        _dec_body,
        grid=grid,
        in_specs=[
            pl.BlockSpec((_BM, _H), row),
            pl.BlockSpec((_BM, _H), row),
            pl.BlockSpec((_H, _H), full),
            pl.BlockSpec((_H, _H), full),
            pl.BlockSpec((1, _H), full),
            pl.BlockSpec((_H, _H), full),
            pl.BlockSpec((1, _H), full),
        ],
        out_specs=pl.BlockSpec((_BM, _H), row),
        out_shape=jax.ShapeDtypeStruct((n, _H), jnp.float32),
    )(x, cx, w0a, w0x, b0, w1, b1)


def _sc_mesh():
    return plsc.VectorSubcoreMesh(core_axis_name="c", subcore_axis_name="s")


def _rsqrt_nr(x):
    i = lax.bitcast_convert_type(x, jnp.int32)
    y = lax.bitcast_convert_type(jnp.int32(0x5F3759DF) - (i >> 1),
                                 jnp.float32)
    for _ in range(3):
        y = y * (1.5 - 0.5 * x * y * y)
    return y


def _sc_edge_feats(tab, s2d, r2d):
    n_ch = s2d.shape[1]
    ne = _NW * n_ch * _CH
    per_tile = n_ch * _CH
    d = tab.shape[1]
    assert n_ch % 2 == 0

    @functools.partial(
        pl.kernel,
        out_type=jax.ShapeDtypeStruct((ne // 8, _H), jnp.float32),
        mesh=_sc_mesh(),
        scratch_types=[
            pltpu.VMEM((n_ch, _CH), jnp.int32),
            pltpu.VMEM((n_ch, _CH), jnp.int32),
            pltpu.VMEM((2, _CH, d), jnp.float32),
            pltpu.VMEM((2, _CH, d), jnp.float32),
            pltpu.VMEM((_CH // 8, _H), jnp.float32),
            pltpu.SemaphoreType.DMA,
            pltpu.SemaphoreType.DMA,
        ],
        compiler_params=pltpu.CompilerParams(use_tc_tiling_on_sc=False),
    )
    def k(tab_h, s_h, r_h, out_h, sidx, ridx, bs, br, ob, sem0, sem1):
        w = lax.axis_index("c") * _NTILE + lax.axis_index("s")
        base = w * per_tile
        sems = (sem0, sem1)

        pltpu.sync_copy(s_h.at[w], sidx)
        pltpu.sync_copy(r_h.at[w], ridx)

        def fire(j, slot):
            pltpu.async_copy(tab_h.at[sidx.at[j]], bs.at[slot], sems[slot])
            pltpu.async_copy(tab_h.at[ridx.at[j]], br.at[slot], sems[slot])

        def drain(j, slot):
            pltpu.make_async_copy(tab_h.at[sidx.at[j]], bs.at[slot],
                                  sems[slot]).wait()
            pltpu.make_async_copy(tab_h.at[ridx.at[j]], br.at[slot],
                                  sems[slot]).wait()

        lanes = lax.iota(jnp.int32, 16)
        geom = lanes <= 8
        dd = (lanes == 9) | (lanes == 10)
        zero16 = jnp.zeros((16,), jnp.float32)
        three = jnp.full((16,), 3, jnp.int32)
        is9 = lanes == 9
        is10 = lanes == 10

        def perm(a, b):
            return jnp.where(is9, jnp.full((16,), a, jnp.int32),
                             jnp.where(is10, jnp.full((16,), b, jnp.int32),
                                       three))

        p1 = perm(0, 4)
        p2 = perm(1, 5)
        p3 = perm(2, 6)
        eps1 = jnp.where(dd, jnp.full((16,), 1e-12, jnp.float32),
                         jnp.full((16,), 1.0, jnp.float32))

        def compute_out(j, slot):
            @plsc.parallel_loop(0, _CH, unroll=2)
            def rowfn(i):
                dp = bs[slot, i, :] - br[slot, i, :]
                sq = dp * dp
                v = (sq.at[p1].get(mode='promise_in_bounds')
                     + sq.at[p2].get(mode='promise_in_bounds')
                     + sq.at[p3].get(mode='promise_in_bounds') + eps1)
                dn = v * _rsqrt_nr(v)
                erow = (jnp.where(geom, dp, zero16)
                        + jnp.where(dd, dn, zero16))
                ob[i >> 3, pl.ds((i & 7) * 16, 16)] = erow

            off8 = pl.multiple_of((base + j * _CH) // 8, _CH // 8)
            pltpu.sync_copy(ob, out_h.at[pl.ds(off8, _CH // 8)])

        fire(0, 0)

        def body(i, carry):
            j0 = 2 * i
            fire(j0 + 1, 1)
            drain(j0, 0)
            compute_out(j0, 0)

            @pl.when(i + 1 < n_ch // 2)
            def _():
                fire(j0 + 2, 0)

            drain(j0 + 1, 1)
            compute_out(j0 + 1, 1)
            return carry

        lax.fori_loop(0, n_ch // 2, body, 0)

    return k(tab, s2d, r2d)


def _sc_cross_gather(ab, s2d, r2d):
    n_ch = s2d.shape[1]
    ne = _NW * n_ch * _CH
    per_tile = n_ch * _CH

    @functools.partial(
        pl.kernel,
        out_type=(jax.ShapeDtypeStruct((ne, _H), jnp.float32),
                  jax.ShapeDtypeStruct((ne, _H), jnp.float32)),
        mesh=_sc_mesh(),
        scratch_types=[
            pltpu.VMEM((n_ch, _CH), jnp.int32),
            pltpu.VMEM((n_ch, _CH), jnp.int32),
            pltpu.VMEM((2, _GCH, 2 * _H), jnp.float32),
            pltpu.VMEM((2, _GCH, 2 * _H), jnp.float32),
            pltpu.VMEM((_GCH, _H), jnp.float32),
            pltpu.VMEM((_GCH, _H), jnp.float32),
            pltpu.SemaphoreType.DMA,
            pltpu.SemaphoreType.DMA,
        ],
    )
    def k(ab_h, s_h, r_h, gm_h, gu_h, sidx, ridx, ts, tr, gm, gu, sem0, sem1):
        w = lax.axis_index("c") * _NTILE + lax.axis_index("s")
        base = w * per_tile
        sems = (sem0, sem1)

        pltpu.sync_copy(s_h.at[w], sidx)
        pltpu.sync_copy(r_h.at[w], ridx)

        def fire(row, half, slot):
            si = sidx.at[row, pl.ds(half * _GCH, _GCH)]
            ri = ridx.at[row, pl.ds(half * _GCH, _GCH)]
            pltpu.async_copy(ab_h.at[si], ts.at[slot], sems[slot])
            pltpu.async_copy(ab_h.at[ri], tr.at[slot], sems[slot])

        def drain(row, half, slot):
            si = sidx.at[row, pl.ds(half * _GCH, _GCH)]
            ri = ridx.at[row, pl.ds(half * _GCH, _GCH)]
            pltpu.make_async_copy(ab_h.at[si], ts.at[slot], sems[slot]).wait()
            pltpu.make_async_copy(ab_h.at[ri], tr.at[slot], sems[slot]).wait()

        def compute_out(row, half, slot):
            @plsc.parallel_loop(0, _GCH, unroll=4)
            def rowfn(i):
                for gidx in range(_H // 16):
                    sl = pl.ds(gidx * 16, 16)
                    sh = pl.ds(_H + gidx * 16, 16)
                    gm[i, sl] = tr[slot, i, sl] + ts[slot, i, sh]
                    gu[i, sl] = ts[slot, i, sl] + tr[slot, i, sh]

            off = pl.multiple_of(base + row * _CH + half * _GCH, _GCH)
            pltpu.sync_copy(gm, gm_h.at[pl.ds(off, _GCH)])
            pltpu.sync_copy(gu, gu_h.at[pl.ds(off, _GCH)])

        fire(0, 0, 0)

        def body(i, carry):
            fire(i, 1, 1)
            drain(i, 0, 0)
            compute_out(i, 0, 0)

            @pl.when(i + 1 < n_ch)
            def _():
                fire(i + 1, 0, 0)

            drain(i, 1, 1)
            compute_out(i, 1, 1)
            return carry

        lax.fori_loop(0, n_ch, body, 0)

    return k(ab, s2d, r2d)


def _sc_segsum(msg, r2d, nacc):
    n_ch = r2d.shape[1]
    ne = _NW * n_ch * _CH
    per_tile = n_ch * _CH
    rows_per_tile = nacc // _NTILE
    n_zch = rows_per_tile // _CH

    @functools.partial(
        pl.kernel,
        out_type=jax.ShapeDtypeStruct((_NSC, nacc, _H), jnp.float32),
        mesh=_sc_mesh(),
        scratch_types=[
            pltpu.VMEM((2, _CH, _H), jnp.float32),
            pltpu.VMEM((n_ch, _CH), jnp.int32),
            pltpu.VMEM_SHARED((nacc, _H), jnp.float32),
            pltpu.SemaphoreType.DMA,
            pltpu.SemaphoreType.DMA,
        ],
    )
    def k(msg_h, r2d_h, out_h, mbuf, idxb, acc, sem0, sem1):
        c = lax.axis_index("c")
        s = lax.axis_index("s")
        w = c * _NTILE + s
        sems = (sem0, sem1)

        @plsc.parallel_loop(0, _CH)
        def zrow(i):
            for gi in range(_H // 16):
                mbuf[0, i, pl.ds(gi * 16, 16)] = jnp.zeros((16,), jnp.float32)

        def zc(j, carry):
            pltpu.sync_copy(mbuf.at[0],
                            acc.at[pl.ds(s * rows_per_tile + j * _CH, _CH)])
            return carry

        lax.fori_loop(0, n_zch, zc, 0)
        pltpu.sync_copy(r2d_h.at[w], idxb)
        plsc.subcore_barrier()

        def body(j, carry):
            off = pl.multiple_of(w * per_tile + j * _CH, _CH)
            pltpu.sync_copy(msg_h.at[pl.ds(off, _CH)], mbuf.at[1])
            pltpu.sync_copy(mbuf.at[1], acc.at[idxb.at[j]], add=True)
            return carry

        lax.fori_loop(0, n_ch, body, 0)
        plsc.subcore_barrier()

        def wc(j, carry):
            rows = pl.ds(s * rows_per_tile + j * _CH, _CH)
            pltpu.sync_copy(acc.at[rows], out_h.at[c].at[rows])
            return carry

        lax.fori_loop(0, n_zch, wc, 0)

    return k(msg, r2d)


def kernel(world_pos, mesh_pos, phi, swelling_phi, swelling_phi_rate,
           node_type, time, mat_param, edge_index, coarse_edge_index, params):
    f32 = jnp.float32
    n = world_pos.shape[0]
    e = edge_index.shape[1]
    ce = coarse_edge_index.shape[1]
    blk = _NW * _CH * 2
    np_ = _rup(n, _NTILE * _CH)
    ep = _rup(e, blk)
    cep = _rup(ce, blk)

    t = time[0]
    freqs = 2.0 ** jnp.arange(8, dtype=f32)
    temb = jnp.concatenate([jnp.sin(freqs * t), jnp.cos(freqs * t)])
    x36 = jnp.concatenate([
        world_pos - mesh_pos, phi, swelling_phi, swelling_phi_rate, node_type,
        jnp.tile(temb[None, :], (n, 1)), jnp.tile(time[None, :], (n, 1)),
        jnp.tile(mat_param[None, :], (n, 1))], axis=1)
    x64 = jnp.zeros((np_, 64), f32).at[:n, :36].set(x36)

    p16 = jnp.zeros((np_, 16), f32)
    p16 = p16.at[:n, 0:3].set(mesh_pos).at[:n, 4:7].set(world_pos)
    p16 = p16.at[:n, 8:9].set(phi)

    nsp_f = 2
    nsp_c = 1

    def idx_parts(idx, n_all, nsp, padval, ch):
        flat = _pad_idx(idx, n_all, padval)
        half = n_all // nsp
        sh = (_NW, half // (_NW * ch), ch)
        return [flat[i * half:(i + 1) * half].reshape(sh) for i in range(nsp)]

    sp_f = idx_parts(edge_index[0], ep, nsp_f, n, _CH)
    rp_f = idx_parts(edge_index[1], ep, nsp_f, n, _CH)
    sp_c = idx_parts(coarse_edge_index[0], cep, nsp_c, n, _CH)
    rp_c = idx_parts(coarse_edge_index[1], cep, nsp_c, n, _CH)

    pp = params
    ne_ = pp['node_enc']
    w0n = jnp.pad(ne_['w0'], ((0, 64 - ne_['w0'].shape[0]), (0, 0)))
    x_h = _tc_mlp_ln(x64, w0n, ne_['b0'][None, :], ne_['w1'],
                     ne_['b1'][None, :], ne_['ln_g'][None, :],
                     ne_['ln_b'][None, :])

    def enc_edges(sparts, rparts, penc):
        w0 = penc['w0']
        wmap = jnp.zeros((16, _H), f32)
        wmap = wmap.at[0:3].set(w0[0:3]).at[4:7].set(w0[4:7])
        wmap = wmap.at[8].set(w0[8]).at[9].set(w0[3]).at[10].set(w0[7])
        w0big = jnp.kron(jnp.eye(8, dtype=f32), wmap)
        out = []
        for spart, rpart in zip(sparts, rparts):
            epk = _sc_edge_feats(p16, spart, rpart)
            out.append(_tc_edge_enc(
                epk, w0big, penc['b0'][None, :], penc['w1'],
                penc['b1'][None, :], penc['ln_g'][None, :],
                penc['ln_b'][None, :], epk.shape[0] * 8, 0))
        return out

    e_h = enc_edges(sp_f, rp_f, pp['edge_enc'])
    ce_h = enc_edges(sp_c, rp_c, pp['cedge_enc'])

    def run_scale(x_h, e_h, procs, sp, rp):
        nsp = len(sp)
        for p_ in procs:
            em = p_['edge_mlp']
            nm = p_['node_mlp']
            wsr = jnp.concatenate([em['w0'][0:_H, :], em['w0'][_H:2 * _H, :]],
                                  axis=1)
            ab = _tc_matmul(x_h, wsr)
            gs = [_sc_cross_gather(ab, sp[i], rp[i]) for i in range(nsp)]
            parts = []
            new_e = []
            for i in range(nsp):
                gm, gu = gs[i]
                msg, en = _tc_edge_step(
                    e_h[i], gm, gu, em['w0'][2 * _H:3 * _H, :],
                    em['b0'][None, :], em['w1'], em['b1'][None, :],
                    em['ln_g'][None, :], em['ln_b'][None, :])
                new_e.append(en)
                part = _sc_segsum(msg, rp[i], np_)
                parts.extend([part[0], part[1]])
            e_h = new_e
            x_h = _tc_node_step(
                parts, x_h, nm['w0'][0:_H, :], nm['w0'][_H:2 * _H, :],
                nm['b0'][None, :], nm['w1'], nm['b1'][None, :],
                nm['ln_g'][None, :], nm['ln_b'][None, :])
        return x_h, e_h

    cx_h = x_h
    x_h, _ = run_scale(x_h, e_h, pp['procs'], sp_f, rp_f)
    cx_h, _ = run_scale(cx_h, ce_h, pp['cprocs'], sp_c, rp_c)

    dec = pp['dec']
    w1p = jnp.pad(dec['w1'], ((0, 0), (0, _H - dec['w1'].shape[1])))
    b1p = jnp.pad(dec['b1'], (0, _H - dec['b1'].shape[0]))[None, :]
    out = _tc_decoder(x_h, cx_h, dec['w0'][0:_H, :], dec['w0'][_H:2 * _H, :],
                      dec['b0'][None, :], w1p, b1p)
    return out[:n, :3]


# speedup vs baseline: 1.1593x; 1.1593x over previous
import functools

import jax
import jax.numpy as jnp
from jax import lax
from jax.experimental import pallas as pl
from jax.experimental.pallas import tpu as pltpu
from jax.experimental.pallas import tpu_sc as plsc

_H = 128
_CH = 64
_NSC = 2
_NTILE = 16
_NW = _NSC * _NTILE
_BM = 512


def _rup(n, m):
    return ((n + m - 1) // m) * m


def _pad_idx(a, n, val):
    if a.shape[0] == n:
        return a
    return jnp.concatenate([a, jnp.full((n - a.shape[0],), val, a.dtype)])


def _ln_in(o, g, b):
    mu = jnp.mean(o, axis=-1, keepdims=True)
    d = o - mu
    var = jnp.mean(d * d, axis=-1, keepdims=True)
    return d * lax.rsqrt(var + 1e-5) * g + b


def _dot(a, b):
    return jnp.dot(a, b, preferred_element_type=jnp.float32)


def _mlp_ln_body(x_ref, w0_ref, b0_ref, w1_ref, b1_ref, g_ref, bb_ref, o_ref):
    h = jnp.maximum(_dot(x_ref[...], w0_ref[...]) + b0_ref[...], 0.0)
    o = _dot(h, w1_ref[...]) + b1_ref[...]
    o_ref[...] = _ln_in(o, g_ref[...], bb_ref[...])


def _tc_mlp_ln(x, w0, b0, w1, b1, g, b):
    n, k = x.shape
    grid = (n // _BM,)
    return pl.pallas_call(
        _mlp_ln_body,
        grid=grid,
        in_specs=[
            pl.BlockSpec((_BM, k), lambda i: (i, 0)),
            pl.BlockSpec((k, _H), lambda i: (0, 0)),
            pl.BlockSpec((1, _H), lambda i: (0, 0)),
            pl.BlockSpec((_H, _H), lambda i: (0, 0)),
            pl.BlockSpec((1, _H), lambda i: (0, 0)),
            pl.BlockSpec((1, _H), lambda i: (0, 0)),
            pl.BlockSpec((1, _H), lambda i: (0, 0)),
        ],
        out_specs=pl.BlockSpec((_BM, _H), lambda i: (i, 0)),
        out_shape=jax.ShapeDtypeStruct((n, _H), jnp.float32),
    )(x, w0, b0, w1, b1, g, b)


def _edge_enc_body(ps_ref, pr_ref, wd_ref, w3_ref, w7_ref, b0_ref, w1_ref,
                   b1_ref, g_ref, bb_ref, o_ref):
    dp = ps_ref[...] - pr_ref[...]
    sq = dp * dp
    li = lax.broadcasted_iota(jnp.int32, dp.shape, 1)
    s1 = jnp.sum(jnp.where(li < 3, sq, 0.0), axis=1, keepdims=True)
    s2 = jnp.sum(jnp.where((li >= 3) & (li < 6), sq, 0.0), axis=1, keepdims=True)
    d = jnp.sqrt(s1 + 1e-12)
    dw = jnp.sqrt(s2 + 1e-12)
    h = jnp.maximum(_dot(dp, wd_ref[...]) + d * w3_ref[...] + dw * w7_ref[...]
                    + b0_ref[...], 0.0)
    o = _dot(h, w1_ref[...]) + b1_ref[...]
    o_ref[...] = _ln_in(o, g_ref[...], bb_ref[...])


def _tc_edge_enc(ps, pr, wd, w3, w7, b0, w1, b1, g, b):
    n = ps.shape[0]
    grid = (n // _BM,)
    row = lambda i: (i, 0)
    full = lambda i: (0, 0)
    return pl.pallas_call(
        _edge_enc_body,
        grid=grid,
        in_specs=[
            pl.BlockSpec((_BM, 16), row),
            pl.BlockSpec((_BM, 16), row),
            pl.BlockSpec((16, _H), full),
            pl.BlockSpec((1, _H), full),
            pl.BlockSpec((1, _H), full),
            pl.BlockSpec((1, _H), full),
            pl.BlockSpec((_H, _H), full),
            pl.BlockSpec((1, _H), full),
            pl.BlockSpec((1, _H), full),
            pl.BlockSpec((1, _H), full),
        ],
        out_specs=pl.BlockSpec((_BM, _H), row),
        out_shape=jax.ShapeDtypeStruct((n, _H), jnp.float32),
    )(ps, pr, wd, w3, w7, b0, w1, b1, g, b)


def _matmul_body(x_ref, w_ref, o_ref):
    o_ref[...] = _dot(x_ref[...], w_ref[...])


def _tc_matmul(x, w):
    n, k = x.shape
    m = w.shape[1]
    grid = (n // _BM,)
    return pl.pallas_call(
        _matmul_body,
        grid=grid,
        in_specs=[
            pl.BlockSpec((_BM, k), lambda i: (i, 0)),
            pl.BlockSpec((k, m), lambda i: (0, 0)),
        ],
        out_specs=pl.BlockSpec((_BM, m), lambda i: (i, 0)),
        out_shape=jax.ShapeDtypeStruct((n, m), jnp.float32),
    )(x, w)


def _edge_step_body(e_ref, gm_ref, gu_ref, w0e_ref, b0_ref, w1_ref, b1_ref,
                    g_ref, bb_ref, msg_ref, en_ref):
    e = e_ref[...]
    ew = _dot(e, w0e_ref[...]) + b0_ref[...]
    hm = jnp.maximum(gm_ref[...] + ew, 0.0)
    hu = jnp.maximum(gu_ref[...] + ew, 0.0)
    g = g_ref[...]
    bb = bb_ref[...]
    w1 = w1_ref[...]
    b1 = b1_ref[...]
    msg_ref[...] = _ln_in(_dot(hm, w1) + b1, g, bb)
    en_ref[...] = _ln_in(_dot(hu, w1) + b1, g, bb) + e


def _tc_edge_step(e, gm, gu, w0e, b0, w1, b1, g, b):
    n = e.shape[0]
    grid = (n // _BM,)
    row = lambda i: (i, 0)
    full = lambda i: (0, 0)
    return pl.pallas_call(
        _edge_step_body,
        grid=grid,
        in_specs=[
            pl.BlockSpec((_BM, _H), row),
            pl.BlockSpec((_BM, _H), row),
            pl.BlockSpec((_BM, _H), row),
            pl.BlockSpec((_H, _H), full),
            pl.BlockSpec((1, _H), full),
            pl.BlockSpec((_H, _H), full),
            pl.BlockSpec((1, _H), full),
            pl.BlockSpec((1, _H), full),
            pl.BlockSpec((1, _H), full),
        ],
        out_specs=[pl.BlockSpec((_BM, _H), row), pl.BlockSpec((_BM, _H), row)],
        out_shape=[jax.ShapeDtypeStruct((n, _H), jnp.float32),
                   jax.ShapeDtypeStruct((n, _H), jnp.float32)],
    )(e, gm, gu, w0e, b0, w1, b1, g, b)


def _node_step_body(p0_ref, p1_ref, x_ref, w0a_ref, w0x_ref, b0_ref, w1_ref,
                    b1_ref, g_ref, bb_ref, o_ref):
    x = x_ref[...]
    agg = p0_ref[...] + p1_ref[...]
    h = jnp.maximum(_dot(agg, w0a_ref[...]) + _dot(x, w0x_ref[...])
                    + b0_ref[...], 0.0)
    o = _dot(h, w1_ref[...]) + b1_ref[...]
    o_ref[...] = _ln_in(o, g_ref[...], bb_ref[...]) + x


def _tc_node_step(p0, p1, x, w0a, w0x, b0, w1, b1, g, b):
    n = x.shape[0]
    grid = (n // _BM,)
    row = lambda i: (i, 0)
    full = lambda i: (0, 0)
    return pl.pallas_call(
        _node_step_body,
        grid=grid,
        in_specs=[
            pl.BlockSpec((_BM, _H), row),
            pl.BlockSpec((_BM, _H), row),
            pl.BlockSpec((_BM, _H), row),
            pl.BlockSpec((_H, _H), full),
            pl.BlockSpec((_H, _H), full),
            pl.BlockSpec((1, _H), full),
            pl.BlockSpec((_H, _H), full),
            pl.BlockSpec((1, _H), full),
            pl.BlockSpec((1, _H), full),
            pl.BlockSpec((1, _H), full),
        ],
        out_specs=pl.BlockSpec((_BM, _H), row),
        out_shape=jax.ShapeDtypeStruct((n, _H), jnp.float32),
    )(p0, p1, x, w0a, w0x, b0, w1, b1, g, b)


def _dec_body(x_ref, cx_ref, w0a_ref, w0x_ref, b0_ref, w1_ref, b1_ref, o_ref):
    h = jnp.maximum(_dot(x_ref[...], w0a_ref[...])
                    + _dot(cx_ref[...], w0x_ref[...]) + b0_ref[...], 0.0)
    o_ref[...] = _dot(h, w1_ref[...]) + b1_ref[...]


def _tc_decoder(x, cx, w0a, w0x, b0, w1, b1):
    n = x.shape[0]
    grid = (n // _BM,)
    row = lambda i: (i, 0)
    full = lambda i: (0, 0)
    return pl.pallas_call(
        _dec_body,
        grid=grid,
        in_specs=[
            pl.BlockSpec((_BM, _H), row),
            pl.BlockSpec((_BM, _H), row),
            pl.BlockSpec((_H, _H), full),
            pl.BlockSpec((_H, _H), full),
            pl.BlockSpec((1, _H), full),
            pl.BlockSpec((_H, _H), full),
            pl.BlockSpec((1, _H), full),
        ],
        out_specs=pl.BlockSpec((_BM, _H), row),
        out_shape=jax.ShapeDtypeStruct((n, _H), jnp.float32),
    )(x, cx, w0a, w0x, b0, w1, b1)


def _sc_mesh():
    return plsc.VectorSubcoreMesh(core_axis_name="c", subcore_axis_name="s")


def _sc_gather_pair(tab, s_idx, r_idx):
    ne = s_idx.shape[0]
    d = tab.shape[1]
    per_tile = ne // _NW
    n_ch = per_tile // _CH

    @functools.partial(
        pl.kernel,
        out_type=(jax.ShapeDtypeStruct((ne, d), jnp.float32),
                  jax.ShapeDtypeStruct((ne, d), jnp.float32)),
        mesh=_sc_mesh(),
        scratch_types=[
            pltpu.VMEM((_CH,), jnp.int32),
            pltpu.VMEM((_CH,), jnp.int32),
            pltpu.VMEM((_CH, d), jnp.float32),
            pltpu.VMEM((_CH, d), jnp.float32),
            pltpu.SemaphoreType.DMA,
        ],
        compiler_params=pltpu.CompilerParams(use_tc_tiling_on_sc=False),
    )
    def k(tab_h, s_h, r_h, ps_h, pr_h, sbuf, rbuf, bs, br, sem):
        w = lax.axis_index("c") * _NTILE + lax.axis_index("s")
        base = w * per_tile

        def body(j, carry):
            off = pl.multiple_of(base + j * _CH, _CH)
            pltpu.sync_copy(s_h.at[pl.ds(off, _CH)], sbuf)
            pltpu.sync_copy(r_h.at[pl.ds(off, _CH)], rbuf)
            cs = pltpu.async_copy(tab_h.at[sbuf], bs, sem)
            cr = pltpu.async_copy(tab_h.at[rbuf], br, sem)
            cs.wait()
            cr.wait()
            pltpu.sync_copy(bs, ps_h.at[pl.ds(off, _CH)])
            pltpu.sync_copy(br, pr_h.at[pl.ds(off, _CH)])
            return carry

        lax.fori_loop(0, n_ch, body, 0)

    return k(tab, s_idx, r_idx)


def _sc_cross_gather(ab, s2d, r2d):
    n_ch = s2d.shape[1]
    ne = _NW * n_ch * _CH
    per_tile = n_ch * _CH
    assert n_ch % 2 == 0

    @functools.partial(
        pl.kernel,
        out_type=(jax.ShapeDtypeStruct((ne, _H), jnp.float32),
                  jax.ShapeDtypeStruct((ne, _H), jnp.float32)),
        mesh=_sc_mesh(),
        scratch_types=[
            pltpu.VMEM((n_ch, _CH), jnp.int32),
            pltpu.VMEM((n_ch, _CH), jnp.int32),
            pltpu.VMEM((2, _CH, 2 * _H), jnp.float32),
            pltpu.VMEM((2, _CH, 2 * _H), jnp.float32),
            pltpu.VMEM((_CH, _H), jnp.float32),
            pltpu.VMEM((_CH, _H), jnp.float32),
            pltpu.SemaphoreType.DMA,
            pltpu.SemaphoreType.DMA,
        ],
    )
    def k(ab_h, s_h, r_h, gm_h, gu_h, sidx, ridx, ts, tr, gm, gu, sem0, sem1):
        w = lax.axis_index("c") * _NTILE + lax.axis_index("s")
        base = w * per_tile
        sems = (sem0, sem1)

        pltpu.sync_copy(s_h.at[w], sidx)
        pltpu.sync_copy(r_h.at[w], ridx)

        def fire(j, slot):
            pltpu.async_copy(ab_h.at[sidx.at[j]], ts.at[slot], sems[slot])
            pltpu.async_copy(ab_h.at[ridx.at[j]], tr.at[slot], sems[slot])

        def drain(j, slot):
            pltpu.make_async_copy(ab_h.at[sidx.at[j]], ts.at[slot],
                                  sems[slot]).wait()
            pltpu.make_async_copy(ab_h.at[ridx.at[j]], tr.at[slot],
                                  sems[slot]).wait()

        def compute_out(j, slot):
            @plsc.parallel_loop(0, _CH, unroll=4)
            def rowfn(i):
                for gidx in range(_H // 16):
                    sl = pl.ds(gidx * 16, 16)
                    sh = pl.ds(_H + gidx * 16, 16)
                    gm[i, sl] = tr[slot, i, sl] + ts[slot, i, sh]
                    gu[i, sl] = ts[slot, i, sl] + tr[slot, i, sh]

            off = pl.multiple_of(base + j * _CH, _CH)
            pltpu.sync_copy(gm, gm_h.at[pl.ds(off, _CH)])
            pltpu.sync_copy(gu, gu_h.at[pl.ds(off, _CH)])

        fire(0, 0)

        def body(i, carry):
            j0 = 2 * i
            fire(j0 + 1, 1)
            drain(j0, 0)
            compute_out(j0, 0)

            @pl.when(i + 1 < n_ch // 2)
            def _():
                fire(j0 + 2, 0)

            drain(j0 + 1, 1)
            compute_out(j0 + 1, 1)
            return carry

        lax.fori_loop(0, n_ch // 2, body, 0)

    return k(ab, s2d, r2d)


def _sc_segsum(msg, r2d, nacc):
    n_ch = r2d.shape[1]
    ne = _NW * n_ch * _CH
    per_tile = n_ch * _CH
    rows_per_tile = nacc // _NTILE
    n_zch = rows_per_tile // _CH

    @functools.partial(
        pl.kernel,
        out_type=jax.ShapeDtypeStruct((_NSC, nacc, _H), jnp.float32),
        mesh=_sc_mesh(),
        scratch_types=[
            pltpu.VMEM((_CH, _H), jnp.float32),
            pltpu.VMEM((_CH, _H), jnp.float32),
            pltpu.VMEM((n_ch, _CH), jnp.int32),
            pltpu.VMEM_SHARED((nacc, _H), jnp.float32),
            pltpu.SemaphoreType.DMA,
        ],
    )
    def k(msg_h, r2d_h, out_h, zbuf, mbuf, idxb, acc, sem):
        c = lax.axis_index("c")
        s = lax.axis_index("s")
        w = c * _NTILE + s

        def zrow(i, carry):
            for gidx in range(_H // 16):
                zbuf[i, pl.ds(gidx * 16, 16)] = jnp.zeros((16,), jnp.float32)
            return carry

        lax.fori_loop(0, _CH, zrow, 0)

        def zc(j, carry):
            pltpu.sync_copy(zbuf, acc.at[pl.ds(s * rows_per_tile + j * _CH,
                                               _CH)])
            return carry

        lax.fori_loop(0, n_zch, zc, 0)
        plsc.subcore_barrier()

        pltpu.sync_copy(r2d_h.at[w], idxb)

        def body(j, carry):
            off = pl.multiple_of(w * per_tile + j * _CH, _CH)
            pltpu.sync_copy(msg_h.at[pl.ds(off, _CH)], mbuf)
            pltpu.sync_copy(mbuf, acc.at[idxb.at[j]], add=True)
            return carry

        lax.fori_loop(0, n_ch, body, 0)
        plsc.subcore_barrier()

        def wc(j, carry):
            rows = pl.ds(s * rows_per_tile + j * _CH, _CH)
            pltpu.sync_copy(acc.at[rows], out_h.at[c].at[rows])
            return carry

        lax.fori_loop(0, n_zch, wc, 0)

    return k(msg, r2d)


def _prep_edge_enc(p):
    w0 = p['w0']
    wd = jnp.zeros((16, _H), jnp.float32)
    wd = wd.at[0:3].set(w0[0:3])
    wd = wd.at[3:6].set(w0[4:7])
    wd = wd.at[6].set(w0[8])
    return (wd, w0[3:4], w0[7:8], p['b0'][None, :], p['w1'], p['b1'][None, :],
            p['ln_g'][None, :], p['ln_b'][None, :])


def kernel(world_pos, mesh_pos, phi, swelling_phi, swelling_phi_rate,
           node_type, time, mat_param, edge_index, coarse_edge_index, params):
    f32 = jnp.float32
    n = world_pos.shape[0]
    e = edge_index.shape[1]
    ce = coarse_edge_index.shape[1]
    blk = _NW * _CH * 2
    np_ = _rup(n, _NW * _CH)
    ep = _rup(e, blk)
    cep = _rup(ce, blk)

    t = time[0]
    freqs = 2.0 ** jnp.arange(8, dtype=f32)
    temb = jnp.concatenate([jnp.sin(freqs * t), jnp.cos(freqs * t)])
    x36 = jnp.concatenate([
        world_pos - mesh_pos, phi, swelling_phi, swelling_phi_rate, node_type,
        jnp.tile(temb[None, :], (n, 1)), jnp.tile(time[None, :], (n, 1)),
        jnp.tile(mat_param[None, :], (n, 1))], axis=1)
    x64 = jnp.zeros((np_, 64), f32).at[:n, :36].set(x36)

    p16 = jnp.zeros((np_, 16), f32)
    p16 = p16.at[:n, 0:3].set(mesh_pos).at[:n, 3:6].set(world_pos)
    p16 = p16.at[:n, 6:7].set(phi)

    s_f = _pad_idx(edge_index[0], ep, 0)
    r_f = _pad_idx(edge_index[1], ep, 0)
    s_c = _pad_idx(coarse_edge_index[0], cep, 0)
    r_c = _pad_idx(coarse_edge_index[1], cep, 0)
    sh_f = (_NW, ep // (_NW * _CH), _CH)
    sh_c = (_NW, cep // (_NW * _CH), _CH)
    s2d_f = s_f.reshape(sh_f)
    r2d_f = r_f.reshape(sh_f)
    s2d_c = s_c.reshape(sh_c)
    r2d_c = r_c.reshape(sh_c)
    rsc_f = _pad_idx(edge_index[1], ep, n).reshape(sh_f)
    rsc_c = _pad_idx(coarse_edge_index[1], cep, n).reshape(sh_c)

    pp = params
    ne_ = pp['node_enc']
    w0n = jnp.pad(ne_['w0'], ((0, 64 - ne_['w0'].shape[0]), (0, 0)))
    x_h = _tc_mlp_ln(x64, w0n, ne_['b0'][None, :], ne_['w1'],
                     ne_['b1'][None, :], ne_['ln_g'][None, :],
                     ne_['ln_b'][None, :])

    ps, pr = _sc_gather_pair(p16, s_f, r_f)
    e_h = _tc_edge_enc(ps, pr, *_prep_edge_enc(pp['edge_enc']))
    cps, cpr = _sc_gather_pair(p16, s_c, r_c)
    ce_h = _tc_edge_enc(cps, cpr, *_prep_edge_enc(pp['cedge_enc']))

    def run_scale(x_h, e_h, procs, s2d, r2d, rsc):
        for p_ in procs:
            em = p_['edge_mlp']
            nm = p_['node_mlp']
            wsr = jnp.concatenate([em['w0'][0:_H, :], em['w0'][_H:2 * _H, :]],
                                  axis=1)
            ab = _tc_matmul(x_h, wsr)
            gm, gu = _sc_cross_gather(ab, s2d, r2d)
            msg, e_h = _tc_edge_step(
                e_h, gm, gu, em['w0'][2 * _H:3 * _H, :], em['b0'][None, :],
                em['w1'], em['b1'][None, :], em['ln_g'][None, :],
                em['ln_b'][None, :])
            part = _sc_segsum(msg, rsc, np_)
            x_h = _tc_node_step(
                part[0], part[1], x_h, nm['w0'][0:_H, :], nm['w0'][_H:2 * _H, :],
                nm['b0'][None, :], nm['w1'], nm['b1'][None, :],
                nm['ln_g'][None, :], nm['ln_b'][None, :])
        return x_h, e_h

    cx_h = x_h
    x_h, e_h = run_scale(x_h, e_h, pp['procs'], s2d_f, r2d_f, rsc_f)
    cx_h, ce_h = run_scale(cx_h, ce_h, pp['cprocs'], s2d_c, r2d_c, rsc_c)

    dec = pp['dec']
    w1p = jnp.pad(dec['w1'], ((0, 0), (0, _H - dec['w1'].shape[1])))
    b1p = jnp.pad(dec['b1'], (0, _H - dec['b1'].shape[0]))[None, :]
    out = _tc_decoder(x_h, cx_h, dec['w0'][0:_H, :], dec['w0'][_H:2 * _H, :],
                      dec['b0'][None, :], w1p, b1p)
    return out[:n, :3]
